# Initial kernel scaffold; baseline (speedup 1.0000x reference)
import jax
import jax.numpy as jnp
from jax.experimental import pallas as pl


def kernel(feature, W1, b1, W2, b2, W3, b3):
    raise NotImplementedError("write your pallas kernel here")


# speedup vs baseline: 1.5564x; 1.5564x over previous
import numpy as np

import jax
import jax.numpy as jnp
from jax.experimental import pallas as pl
from jax.experimental.pallas import tpu as pltpu

N = 5000
NP = 5120
BN = 512
NB = NP // BN
NT = NB * (NB + 1) // 2
D_IN = 500
DP = 512
H = 256

f32 = jnp.float32
bf16 = jnp.bfloat16
NEG = -3.0e38
POS = 3.0e38

_A_OF = np.concatenate([np.full(a + 1, a, np.int32) for a in range(NB)])
_B_OF = np.concatenate([np.arange(a + 1, dtype=np.int32) for a in range(NB)])


def _fused_kernel(amap_ref, bmap_ref,
                  fa_ref, fb_ref, sqc_ref, sqr_ref,
                  f32_ref, w1_ref, w2_ref, w3_ref,
                  bs_ref, tri_ref, trif_ref,
                  pool_ref,
                  e_scr, g_scr, za_scr, zb_scr, deg_scr, acc_scr,
                  mxg_smem, mn_smem, mxt_smem, cnt_smem):
    p = pl.program_id(0)
    t = pl.program_id(1)
    a = amap_ref[t]
    b = bmap_ref[t]

    def gram_tile(ra, rb):
        return jax.lax.dot_general(
            fa_ref[pl.ds(ra * BN, BN), :], fb_ref[:, pl.ds(rb * BN, BN)],
            dimension_numbers=(((1,), (0,)), ((), ())),
            preferred_element_type=f32)

    def add_sq(g, ra, rb):
        return ((g + sqc_ref[pl.ds(ra * BN, BN), 0:1])
                + sqr_ref[0:1, pl.ds(rb * BN, BN)])

    def reduce_prev(pt):
        pa = amap_ref[pt]
        pb = bmap_ref[pt]
        g = add_sq(g_scr[pt % 2], pa, pb)

        @pl.when(pb < pa)
        def _():
            mx = jnp.max(jnp.max(g, axis=0, keepdims=True))
            mn = jnp.min(jnp.min(g, axis=0, keepdims=True))
            mn_smem[pt] = mn
            mxt_smem[pt] = mx
            mxg_smem[0] = jnp.maximum(mxg_smem[0], mx)

        @pl.when(pb == pa)
        def _():
            trim = trif_ref[...] != 0.0
            mx = jnp.max(jnp.max(jnp.where(trim, g, NEG),
                                 axis=0, keepdims=True))
            mn = jnp.min(jnp.min(jnp.where(trim, g, POS),
                                 axis=0, keepdims=True))
            mn_smem[pt] = mn
            mxt_smem[pt] = mx
            mxg_smem[0] = jnp.maximum(mxg_smem[0], mx)

    @pl.when(p == 0)
    def _():
        @pl.when(t == 0)
        def _():
            mxg_smem[0] = NEG

        g_scr[t % 2] = gram_tile(a, b)

        @pl.when(t > 0)
        def _():
            reduce_prev(t - 1)

    @pl.when(p == 1)
    def _():
        @pl.when(t == 0)
        def _():
            reduce_prev(NT - 1)

        thr = 0.5 * jnp.maximum(mxg_smem[0], 0.0)
        mn = mn_smem[t]
        mxt = mxt_smem[t]

        @pl.when(b == 0)
        def _():
            rows = a * BN + jax.lax.broadcasted_iota(jnp.int32, (BN, 1), 0)
            deg_scr[pl.ds(a * BN, BN), :] = jnp.broadcast_to(
                jnp.where(rows < N, 1.0, 0.0).astype(f32), (BN, 128))

        @pl.when(mn >= thr)
        def _():
            cnt_smem[t] = 0.0

        @pl.when((mn < thr) & (mxt < thr))
        def _():
            @pl.when(b < a)
            def _():
                e_scr[t] = jnp.ones((BN, BN), jnp.int8)
                deg_scr[pl.ds(a * BN, BN), :] += float(BN)
                cnt_smem[t] = float(BN * BN)

            @pl.when(b == a)
            def _():
                e_scr[t] = tri_ref[...]
                li = jax.lax.broadcasted_iota(jnp.int32, (BN, 1), 0)
                deg_scr[pl.ds(a * BN, BN), :] += jnp.broadcast_to(
                    li.astype(f32), (BN, 128))
                cnt_smem[t] = float(BN * (BN - 1) // 2)

        @pl.when((mn < thr) & (mxt >= thr))
        def _():
            dist = add_sq(gram_tile(a, b), a, b)
            mask = dist < thr

            @pl.when(b == a)
            def _():
                mask2 = mask & (trif_ref[...] != 0.0)
                e_f = jnp.where(mask2, 1.0, 0.0).astype(f32)
                e_scr[t] = e_f.astype(jnp.int8)
                rs = jnp.dot(e_f.astype(bf16), jnp.ones((BN, 128), bf16),
                             preferred_element_type=f32)
                deg_scr[pl.ds(a * BN, BN), :] += rs
                cnt_smem[t] = jnp.sum(rs[:, 0:1])

            @pl.when(b < a)
            def _():
                e_f = jnp.where(mask, 1.0, 0.0).astype(f32)
                e_scr[t] = e_f.astype(jnp.int8)
                rs = jnp.dot(e_f.astype(bf16), jnp.ones((BN, 128), bf16),
                             preferred_element_type=f32)
                deg_scr[pl.ds(a * BN, BN), :] += rs
                cnt_smem[t] = jnp.sum(rs[:, 0:1])

        @pl.when(b == a)
        def _():
            deg = deg_scr[pl.ds(a * BN, BN), 0:1]
            dis = jnp.where(deg > 0.0, jax.lax.rsqrt(deg), 0.0)
            za_scr[pl.ds(a * BN, BN), :] = dis * jnp.dot(
                f32_ref[...], w1_ref[...], preferred_element_type=f32)

    def layer(l, zin, zout, wn):
        jd = a
        isrc = b

        @pl.when(isrc == 0)
        def _():
            if zout is None:
                acc_scr[...] = jnp.zeros((BN, H), f32)
            else:
                zout[pl.ds(jd * BN, BN), :] = jnp.zeros((BN, H), f32)

        @pl.when(t == 0)
        def _():
            pool_ref[pl.ds(l * 8, 8), :] = jnp.zeros((8, H), f32)

        @pl.when(cnt_smem[t] > 0.0)
        def _():
            zi = zin[pl.ds(isrc * BN, BN), :].astype(bf16)
            contrib = jnp.dot(e_scr[t].astype(bf16), zi,
                              preferred_element_type=f32)
            if zout is None:
                acc_scr[...] += contrib
            else:
                zout[pl.ds(jd * BN, BN), :] += contrib

        @pl.when(isrc == jd)
        def _():
            deg = deg_scr[pl.ds(jd * BN, BN), 0:1]
            dis = jnp.where(deg > 0.0, jax.lax.rsqrt(deg), 0.0)
            acc = (acc_scr[...] if zout is None
                   else zout[pl.ds(jd * BN, BN), :])
            res = (dis * (zin[pl.ds(jd * BN, BN), :] + acc)
                   + bs_ref[pl.ds(l * 8, 8), :][0:1, :])
            res = jnp.maximum(res, 0.0)
            gd = jd * BN + jax.lax.broadcasted_iota(jnp.int32, (BN, 1), 0)
            res = jnp.where(gd < N, res, 0.0)
            cur = pool_ref[pl.ds(l * 8, 8), :]
            pool_ref[pl.ds(l * 8, 8), :] = cur + jnp.broadcast_to(
                jnp.sum(res, axis=0, keepdims=True), (8, H))
            if zout is not None:
                zout[pl.ds(jd * BN, BN), :] = dis * jnp.dot(
                    res, wn, preferred_element_type=f32)

    @pl.when(p == 2)
    def _():
        layer(0, za_scr, zb_scr, w2_ref[...])

    @pl.when(p == 3)
    def _():
        layer(1, zb_scr, za_scr, w3_ref[...])

    @pl.when(p == 4)
    def _():
        layer(2, za_scr, None, None)


def kernel(feature, W1, b1, W2, b2, W3, b3):
    fpad = (jnp.zeros((NP, DP), f32)
            .at[:N, :D_IN].set(feature)
            .at[N:, :D_IN].set(jnp.broadcast_to(feature[0:1],
                                                (NP - N, D_IN))))
    sq = jnp.sum(fpad * fpad, axis=1)
    fa = (-2.0 * fpad).astype(bf16)
    fbt = fpad.astype(bf16).T
    sqc = jnp.broadcast_to(sq[:, None], (NP, 128))
    sqr = jnp.broadcast_to(sq[None, :], (8, NP))
    w1p = jnp.zeros((DP, H), f32).at[:D_IN, :].set(W1)
    tri = jnp.tri(BN, BN, -1, dtype=jnp.int8)
    bs = jnp.concatenate([
        jnp.broadcast_to(b1[None, :], (8, H)),
        jnp.broadcast_to(b2[None, :], (8, H)),
        jnp.broadcast_to(b3[None, :], (8, H))], axis=0)
    amap = jnp.asarray(_A_OF)
    bmap = jnp.asarray(_B_OF)

    pool = pl.pallas_call(
        _fused_kernel,
        grid_spec=pltpu.PrefetchScalarGridSpec(
            num_scalar_prefetch=2,
            grid=(5, NT),
            in_specs=[
                pl.BlockSpec((NP, DP), lambda p, t, am, bm: (0, 0)),
                pl.BlockSpec((DP, NP), lambda p, t, am, bm: (0, 0)),
                pl.BlockSpec((NP, 128), lambda p, t, am, bm: (0, 0)),
                pl.BlockSpec((8, NP), lambda p, t, am, bm: (0, 0)),
                pl.BlockSpec((BN, DP),
                             lambda p, t, am, bm:
                             (jnp.where(p == 1, am[t], 0), 0)),
                pl.BlockSpec((DP, H), lambda p, t, am, bm: (0, 0)),
                pl.BlockSpec((H, H), lambda p, t, am, bm: (0, 0)),
                pl.BlockSpec((H, H), lambda p, t, am, bm: (0, 0)),
                pl.BlockSpec((24, H), lambda p, t, am, bm: (0, 0)),
                pl.BlockSpec((BN, BN), lambda p, t, am, bm: (0, 0)),
                pl.BlockSpec((BN, BN), lambda p, t, am, bm: (0, 0)),
            ],
            out_specs=pl.BlockSpec((24, H), lambda p, t, am, bm: (0, 0)),
            scratch_shapes=[
                pltpu.VMEM((NT, BN, BN), jnp.int8),
                pltpu.VMEM((2, BN, BN), f32),
                pltpu.VMEM((NP, H), f32),
                pltpu.VMEM((NP, H), f32),
                pltpu.VMEM((NP, 128), f32),
                pltpu.VMEM((BN, H), f32),
                pltpu.SMEM((1,), f32),
                pltpu.SMEM((NT,), f32),
                pltpu.SMEM((NT,), f32),
                pltpu.SMEM((NT,), f32),
            ],
        ),
        out_shape=jax.ShapeDtypeStruct((24, H), f32),
    )(amap, bmap, fa, fbt, sqc, sqr, fpad, w1p, W2, W3, bs, tri,
      tri.astype(f32))

    s = (pool[0:1, :] + pool[8:9, :] + pool[16:17, :]) / float(N)
    return jnp.concatenate([s, s], axis=1)
